# Initial kernel scaffold; baseline (speedup 1.0000x reference)
import jax
import jax.numpy as jnp
from jax.experimental import pallas as pl


def kernel(x, edge_index, ex, W_lin, b_lin, W_edge, b_edge, root_emb):
    raise NotImplementedError("write your pallas kernel here")


# speedup vs baseline: 3.9912x; 3.9912x over previous
import functools

import jax
import jax.numpy as jnp
from jax import lax
from jax.experimental import pallas as pl
from jax.experimental.pallas import tpu as pltpu
from jax.experimental.pallas import tpu_sc as plsc

N = 10000
E = 160000
D = 256
H = 128
NP = 10240
NC = 2
NS = 16
EPT = E // NS
EB = 80
NCHUNK = EPT // EB
DEG_EPT = E // (NC * NS)
DEG_B = 1000
DEG_NCHUNK = DEG_EPT // DEG_B
RPT = NP // NS


def _sc_mesh():
    return plsc.VectorSubcoreMesh(core_axis_name="c", subcore_axis_name="s",
                                  num_cores=NC, num_subcores=NS)


def _deg_body(src_h, pdeg_h, ones_v, idx_v, z_v, shared_deg):
    c = lax.axis_index("c")
    s = lax.axis_index("s")

    def zfill(i, _):
        z_v[pl.ds(i * 16, 16)] = jnp.zeros((16,), jnp.float32)
        return 0

    lax.fori_loop(0, RPT // 16, zfill, 0)

    def ofill(i, _):
        ones_v[pl.ds(i * 16, 16)] = jnp.ones((16,), jnp.float32)
        return 0

    lax.fori_loop(0, 63, ofill, 0)

    pltpu.sync_copy(z_v, shared_deg.at[pl.ds(s * RPT, RPT)])
    plsc.subcore_barrier()

    def chunk(k, _):
        eb = pl.multiple_of(c * (E // 2) + s * DEG_EPT + k * DEG_B, 8)
        pltpu.sync_copy(src_h.at[pl.ds(eb, DEG_B)], idx_v)
        pltpu.sync_copy(ones_v.at[pl.ds(0, DEG_B)], shared_deg.at[idx_v],
                        add=True)
        return 0

    lax.fori_loop(0, DEG_NCHUNK, chunk, 0)
    plsc.subcore_barrier()

    @pl.when(s == 0)
    def _():
        pltpu.sync_copy(shared_deg, pdeg_h.at[c])


def _deg_kernel(src):
    return pl.kernel(
        _deg_body,
        out_type=jax.ShapeDtypeStruct((NC, NP), jnp.float32),
        mesh=_sc_mesh(),
        scratch_types=[
            pltpu.VMEM((1008,), jnp.float32),
            pltpu.VMEM((DEG_B,), jnp.int32),
            pltpu.VMEM((RPT,), jnp.float32),
            pltpu.VMEM_SHARED((NP,), jnp.float32),
        ],
        compiler_params=pltpu.CompilerParams(needs_layout_passes=False),
    )(src)


def _edge_body(xlh_h, ehh_h, rh_h, norm_h, src_h, dst_h, out_h,
                norm_v, sidx, didx, ne_v, rows, ehv, sem, shared_out):
    c = lax.axis_index("c")
    s = lax.axis_index("s")

    pltpu.sync_copy(norm_h, norm_v)

    def initk(t, _):
        rb = pl.multiple_of(s * RPT + t * EB, 8)
        pltpu.sync_copy(rh_h.at[c, pl.ds(rb, EB)], rows)

        def rowg(g, _):
            nv = norm_v[pl.ds(rb + g * 16, 16)]
            for i16 in range(16):
                i = g * 16 + i16
                sc2 = nv[i16] * nv[i16]
                for j in range(H // 16):
                    sl = pl.ds(j * 16, 16)
                    rows[i, sl] = rows[i, sl] * sc2
            return 0

        lax.fori_loop(0, EB // 16, rowg, 0)
        pltpu.sync_copy(rows, shared_out.at[pl.ds(rb, EB)])
        return 0

    lax.fori_loop(0, RPT // EB, initk, 0)
    plsc.subcore_barrier()

    def chunk(k, _):
        eb = pl.multiple_of(s * EPT + k * EB, 8)
        pltpu.sync_copy(src_h.at[pl.ds(eb, EB)], sidx)
        pltpu.sync_copy(dst_h.at[pl.ds(eb, EB)], didx)
        pltpu.async_copy(xlh_h.at[c].at[sidx], rows, sem).wait()
        pltpu.sync_copy(ehh_h.at[c, pl.ds(eb, EB)], ehv)

        def nloop(i, _):
            sl = pl.ds(i * 16, 16)
            ns = plsc.load_gather(norm_v, [sidx[sl]])
            nd = plsc.load_gather(norm_v, [didx[sl]])
            ne_v[sl] = ns * nd
            return 0

        lax.fori_loop(0, EB // 16, nloop, 0)

        def eg(g, _):
            nev = ne_v[pl.ds(g * 16, 16)]
            for e16 in range(16):
                e = g * 16 + e16
                ne = nev[e16]
                for j in range(H // 16):
                    sl = pl.ds(j * 16, 16)
                    v = rows[e, sl] + ehv[e, sl]
                    rows[e, sl] = jnp.maximum(v, 0.0) * ne
            return 0

        lax.fori_loop(0, EB // 16, eg, 0)
        pltpu.sync_copy(rows, shared_out.at[didx], add=True)
        return 0

    lax.fori_loop(0, NCHUNK, chunk, 0)
    plsc.subcore_barrier()

    rb = pl.multiple_of(s * RPT, 8)
    pltpu.sync_copy(shared_out.at[pl.ds(rb, RPT)],
                    out_h.at[c, pl.ds(rb, RPT)])


def _edge_kernel(xlh, ehh, rh, norm, src, dst):
    return pl.kernel(
        _edge_body,
        out_type=jax.ShapeDtypeStruct((NC, NP, H), jnp.float32),
        mesh=_sc_mesh(),
        scratch_types=[
            pltpu.VMEM((NP,), jnp.float32),
            pltpu.VMEM((EB,), jnp.int32),
            pltpu.VMEM((EB,), jnp.int32),
            pltpu.VMEM((EB,), jnp.float32),
            pltpu.VMEM((EB, H), jnp.float32),
            pltpu.VMEM((EB, H), jnp.float32),
            pltpu.SemaphoreType.DMA,
            pltpu.VMEM_SHARED((NP, H), jnp.float32),
        ],
        compiler_params=pltpu.CompilerParams(needs_layout_passes=False),
    )(xlh, ehh, rh, norm, src, dst)


def _norm_body(pdeg_ref, norm_ref):
    d = pdeg_ref[0] + pdeg_ref[1] + 1.0
    norm_ref[...] = lax.rsqrt(d)


def _norm_kernel(pdeg):
    pdeg2 = pdeg.reshape(NC, NP // 128, 128)
    out = pl.pallas_call(
        _norm_body,
        out_shape=jax.ShapeDtypeStruct((NP // 128, 128), jnp.float32),
    )(pdeg2)
    return out.reshape(NP)


def _dense_body(x_ref, w_ref, b_ref, re_ref, xlh_ref, rh_ref):
    xl = jnp.dot(x_ref[...], w_ref[...],
                 preferred_element_type=jnp.float32) + b_ref[...]
    r = jnp.maximum(xl + re_ref[...], 0.0)
    xlh_ref[0] = xl[:, :H]
    xlh_ref[1] = xl[:, H:]
    rh_ref[0] = r[:, :H]
    rh_ref[1] = r[:, H:]


def _dense_kernel(x_pad, W_lin, b_lin, root_emb):
    blk = 1024
    grid = NP // blk
    return pl.pallas_call(
        _dense_body,
        grid=(grid,),
        in_specs=[
            pl.BlockSpec((blk, D), lambda j: (j, 0)),
            pl.BlockSpec((D, D), lambda j: (0, 0)),
            pl.BlockSpec((1, D), lambda j: (0, 0)),
            pl.BlockSpec((1, D), lambda j: (0, 0)),
        ],
        out_specs=[
            pl.BlockSpec((NC, blk, H), lambda j: (0, j, 0)),
            pl.BlockSpec((NC, blk, H), lambda j: (0, j, 0)),
        ],
        out_shape=[
            jax.ShapeDtypeStruct((NC, NP, H), jnp.float32),
            jax.ShapeDtypeStruct((NC, NP, H), jnp.float32),
        ],
    )(x_pad, W_lin, b_lin.reshape(1, D), root_emb.reshape(1, D))


def _eh_body(ex_ref, w_ref, b_ref, ehh_ref):
    eh = jnp.dot(ex_ref[...], w_ref[...],
                 preferred_element_type=jnp.float32) + b_ref[...]
    ehh_ref[0] = eh[:, :H]
    ehh_ref[1] = eh[:, H:]


def _eh_kernel(ex_pad, W_edge_pad, b_edge):
    blk = 2000
    grid = E // blk
    return pl.pallas_call(
        _eh_body,
        grid=(grid,),
        in_specs=[
            pl.BlockSpec((blk, 8), lambda j: (j, 0)),
            pl.BlockSpec((8, D), lambda j: (0, 0)),
            pl.BlockSpec((1, D), lambda j: (0, 0)),
        ],
        out_specs=pl.BlockSpec((NC, blk, H), lambda j: (0, j, 0)),
        out_shape=jax.ShapeDtypeStruct((NC, E, H), jnp.float32),
    )(ex_pad, W_edge_pad, b_edge.reshape(1, D))


@jax.jit
def kernel(x, edge_index, ex, W_lin, b_lin, W_edge, b_edge, root_emb):
    src = edge_index[0]
    dst = edge_index[1]

    x_pad = jnp.pad(x, ((0, NP - N), (0, 0)))
    ex_pad = jnp.pad(ex, ((0, 0), (0, 1)))
    W_edge_pad = jnp.pad(W_edge, ((0, 1), (0, 0)))

    pdeg = _deg_kernel(src)
    norm = _norm_kernel(pdeg)
    xlh, rh = _dense_kernel(x_pad, W_lin, b_lin, root_emb)
    ehh = _eh_kernel(ex_pad, W_edge_pad, b_edge)
    outh = _edge_kernel(xlh, ehh, rh, norm, src, dst)
    return jnp.concatenate([outh[0, :N], outh[1, :N]], axis=1)


# speedup vs baseline: 6.1959x; 1.5524x over previous
import jax
import jax.numpy as jnp
from jax import lax
from jax.experimental import pallas as pl
from jax.experimental.pallas import tpu as pltpu
from jax.experimental.pallas import tpu_sc as plsc

N = 10000
E = 160000
D = 256
H = 128
NPD = 10240
NC = 2
NS = 16
EPT = E // NS
EB = 80
NCHUNK = EPT // EB
DEG_EPT = E // (NC * NS)
DEG_B = 1000
DEG_NCHUNK = DEG_EPT // DEG_B
DRPT = NPD // NS
RPT = N // NS


def _sc_mesh():
    return plsc.VectorSubcoreMesh(core_axis_name="c", subcore_axis_name="s",
                                  num_cores=NC, num_subcores=NS)


def _deg_body(src_h, pdeg_h, ones_v, idx_v, z_v, shared_deg):
    c = lax.axis_index("c")
    s = lax.axis_index("s")

    def zfill(i, _):
        z_v[pl.ds(i * 16, 16)] = jnp.zeros((16,), jnp.float32)
        return 0

    lax.fori_loop(0, DRPT // 16, zfill, 0)

    def ofill(i, _):
        ones_v[pl.ds(i * 16, 16)] = jnp.ones((16,), jnp.float32)
        return 0

    lax.fori_loop(0, 63, ofill, 0)

    pltpu.sync_copy(z_v, shared_deg.at[pl.ds(s * DRPT, DRPT)])
    plsc.subcore_barrier()

    def chunk(k, _):
        eb = pl.multiple_of(c * (E // 2) + s * DEG_EPT + k * DEG_B, 8)
        pltpu.sync_copy(src_h.at[pl.ds(eb, DEG_B)], idx_v)
        pltpu.sync_copy(ones_v.at[pl.ds(0, DEG_B)], shared_deg.at[idx_v],
                        add=True)
        return 0

    lax.fori_loop(0, DEG_NCHUNK, chunk, 0)
    plsc.subcore_barrier()

    @pl.when(s == 0)
    def _():
        pltpu.sync_copy(shared_deg, pdeg_h.at[c])


def _deg_kernel(src):
    return pl.kernel(
        _deg_body,
        out_type=jax.ShapeDtypeStruct((NC, NPD), jnp.float32),
        mesh=_sc_mesh(),
        scratch_types=[
            pltpu.VMEM((1008,), jnp.float32),
            pltpu.VMEM((DEG_B,), jnp.int32),
            pltpu.VMEM((DRPT,), jnp.float32),
            pltpu.VMEM_SHARED((NPD,), jnp.float32),
        ],
        compiler_params=pltpu.CompilerParams(needs_layout_passes=False),
    )(src)


def _edge_body(xlh_h, ehh_h, rh_h, norm_h, src_h, dst_h, out_h,
               sidx0, sidx1, didx0, didx1, didx_s0, didx_s1,
               ns0, ns1, nd0, nd1, ne_v,
               rows0, rows1, ehv0, ehv1,
               sg0, sg1, se0, se1, ssn0, ssn1, sdn0, sdn1,
               si0, si1, di0, di1, ss0, ss1, shared_out):
    c = lax.axis_index("c")
    s = lax.axis_index("s")

    rb = pl.multiple_of(s * 640, 8)

    @pl.when(s < NS - 1)
    def _():
        pltpu.sync_copy(rh_h.at[c, pl.ds(rb, 640)],
                        shared_out.at[pl.ds(rb, 640)])

    @pl.when(s == NS - 1)
    def _():
        pltpu.sync_copy(rh_h.at[c, pl.ds(rb, N - 640 * (NS - 1))],
                        shared_out.at[pl.ds(rb, N - 640 * (NS - 1))])

    plsc.subcore_barrier()

    bufs = ((sidx0, didx0, didx_s0, ns0, nd0, rows0, ehv0,
             sg0, se0, ssn0, sdn0, si0, di0, ss0),
            (sidx1, didx1, didx_s1, ns1, nd1, rows1, ehv1,
             sg1, se1, ssn1, sdn1, si1, di1, ss1))

    def issue_idx(k, p):
        sidx, didx = bufs[p][0], bufs[p][1]
        si, di = bufs[p][11], bufs[p][12]
        eb = pl.multiple_of(s * EPT + k * EB, 8)
        pltpu.async_copy(src_h.at[pl.ds(eb, EB)], sidx, si)
        pltpu.async_copy(dst_h.at[pl.ds(eb, EB)], didx, di)

    def wait_idx(p):
        sidx, didx = bufs[p][0], bufs[p][1]
        si, di = bufs[p][11], bufs[p][12]
        eb = pl.multiple_of(s * EPT, 8)
        pltpu.make_async_copy(src_h.at[pl.ds(eb, EB)], sidx, si).wait()
        pltpu.make_async_copy(dst_h.at[pl.ds(eb, EB)], didx, di).wait()

    def issue_gathers(p):
        sidx, didx = bufs[p][0], bufs[p][1]
        nsrc, ndst, rows = bufs[p][3], bufs[p][4], bufs[p][5]
        sg, ssn, sdn = bufs[p][7], bufs[p][9], bufs[p][10]
        pltpu.async_copy(xlh_h.at[c].at[sidx], rows, sg)
        pltpu.async_copy(norm_h.at[sidx], nsrc, ssn)
        pltpu.async_copy(norm_h.at[didx], ndst, sdn)

    def wait_gathers(p):
        sidx, didx = bufs[p][0], bufs[p][1]
        nsrc, ndst, rows = bufs[p][3], bufs[p][4], bufs[p][5]
        sg, ssn, sdn = bufs[p][7], bufs[p][9], bufs[p][10]
        pltpu.make_async_copy(xlh_h.at[c].at[sidx], rows, sg).wait()
        pltpu.make_async_copy(norm_h.at[sidx], nsrc, ssn).wait()
        pltpu.make_async_copy(norm_h.at[didx], ndst, sdn).wait()

    def issue_eh(k, p):
        ehv, se = bufs[p][6], bufs[p][8]
        eb = pl.multiple_of(s * EPT + k * EB, 8)
        pltpu.async_copy(ehh_h.at[c, pl.ds(eb, EB)], ehv, se)

    def wait_eh(p):
        ehv, se = bufs[p][6], bufs[p][8]
        eb = pl.multiple_of(s * EPT, 8)
        pltpu.make_async_copy(ehh_h.at[c, pl.ds(eb, EB)], ehv, se).wait()

    def issue_scatter(p):
        didx_s, ehv, ss = bufs[p][2], bufs[p][6], bufs[p][13]
        pltpu.async_copy(ehv, shared_out.at[didx_s], ss, add=True)

    def wait_scatter(p):
        didx_s, ehv, ss = bufs[p][2], bufs[p][6], bufs[p][13]
        pltpu.make_async_copy(ehv, shared_out.at[didx_s], ss).wait()

    def phase(k, p):
        sidx, didx, didx_s = bufs[p][0], bufs[p][1], bufs[p][2]
        nsrc, ndst = bufs[p][3], bufs[p][4]
        rows, ehv = bufs[p][5], bufs[p][6]

        @pl.when(k + 1 <= NCHUNK - 1)
        def _():
            wait_idx(1 - p)
            issue_gathers(1 - p)

        @pl.when(k > 0)
        def _():
            wait_scatter(1 - p)

        @pl.when(k + 1 <= NCHUNK - 1)
        def _():
            issue_eh(k + 1, 1 - p)

        wait_gathers(p)
        wait_eh(p)

        def nloop(i, _):
            sl = pl.ds(i * 16, 16)
            ne_v[sl] = nsrc[sl] * ndst[sl]
            didx_s[sl] = didx[sl]
            return 0

        lax.fori_loop(0, EB // 16, nloop, 0)

        @pl.when(k + 2 <= NCHUNK - 1)
        def _():
            issue_idx(k + 2, p)

        def eg(g, _):
            nev = ne_v[pl.ds(g * 16, 16)]
            for e16 in range(16):
                e = g * 16 + e16
                ne = nev[e16]
                for j in range(H // 16):
                    sl = pl.ds(j * 16, 16)
                    v = rows[e, sl] + ehv[e, sl]
                    ehv[e, sl] = jnp.maximum(v, 0.0) * ne
            return 0

        lax.fori_loop(0, EB // 16, eg, 0)
        issue_scatter(p)

    issue_idx(0, 0)
    issue_idx(1, 1)
    wait_idx(0)
    issue_gathers(0)
    issue_eh(0, 0)

    def pair(i, _):
        phase(2 * i, 0)
        phase(2 * i + 1, 1)
        return 0

    lax.fori_loop(0, (NCHUNK - 1) // 2, pair, 0)
    phase(NCHUNK - 1, 0)
    wait_scatter(0)

    plsc.subcore_barrier()

    @pl.when(s < NS - 1)
    def _():
        pltpu.sync_copy(shared_out.at[pl.ds(rb, 640)],
                        out_h.at[c, pl.ds(rb, 640)])

    @pl.when(s == NS - 1)
    def _():
        pltpu.sync_copy(shared_out.at[pl.ds(rb, N - 640 * (NS - 1))],
                        out_h.at[c, pl.ds(rb, N - 640 * (NS - 1))])


def _edge_kernel(xlh, ehh, rh, norm, src, dst):
    return pl.kernel(
        _edge_body,
        out_type=jax.ShapeDtypeStruct((NC, N, H), jnp.float32),
        mesh=_sc_mesh(),
        scratch_types=[
            pltpu.VMEM((EB,), jnp.int32),
            pltpu.VMEM((EB,), jnp.int32),
            pltpu.VMEM((EB,), jnp.int32),
            pltpu.VMEM((EB,), jnp.int32),
            pltpu.VMEM((EB,), jnp.int32),
            pltpu.VMEM((EB,), jnp.int32),
            pltpu.VMEM((EB,), jnp.float32),
            pltpu.VMEM((EB,), jnp.float32),
            pltpu.VMEM((EB,), jnp.float32),
            pltpu.VMEM((EB,), jnp.float32),
            pltpu.VMEM((EB,), jnp.float32),
            pltpu.VMEM((EB, H), jnp.float32),
            pltpu.VMEM((EB, H), jnp.float32),
            pltpu.VMEM((EB, H), jnp.float32),
            pltpu.VMEM((EB, H), jnp.float32),
            pltpu.SemaphoreType.DMA,
            pltpu.SemaphoreType.DMA,
            pltpu.SemaphoreType.DMA,
            pltpu.SemaphoreType.DMA,
            pltpu.SemaphoreType.DMA,
            pltpu.SemaphoreType.DMA,
            pltpu.SemaphoreType.DMA,
            pltpu.SemaphoreType.DMA,
            pltpu.SemaphoreType.DMA,
            pltpu.SemaphoreType.DMA,
            pltpu.SemaphoreType.DMA,
            pltpu.SemaphoreType.DMA,
            pltpu.SemaphoreType.DMA,
            pltpu.SemaphoreType.DMA,
            pltpu.VMEM_SHARED((N, H), jnp.float32),
        ],
        compiler_params=pltpu.CompilerParams(needs_layout_passes=False),
    )(xlh, ehh, rh, norm, src, dst)


def _norm_body(pdeg_ref, norm_ref, dinv_ref):
    d = pdeg_ref[0] + pdeg_ref[1] + 1.0
    norm_ref[...] = lax.rsqrt(d)
    dinv_ref[...] = 1.0 / d


def _norm_kernel(pdeg):
    pdeg2 = pdeg.reshape(NC, NPD // 128, 128)
    norm, dinv = pl.pallas_call(
        _norm_body,
        out_shape=[
            jax.ShapeDtypeStruct((NPD // 128, 128), jnp.float32),
            jax.ShapeDtypeStruct((NPD // 128, 128), jnp.float32),
        ],
    )(pdeg2)
    return norm.reshape(NPD), dinv.reshape(NPD)


def _dense_body(x_ref, w_ref, b_ref, re_ref, dinv_ref, xlh_ref, rh_ref):
    xl = jnp.dot(x_ref[...], w_ref[...],
                 preferred_element_type=jnp.float32) + b_ref[...]
    r = jnp.maximum(xl + re_ref[...], 0.0) * dinv_ref[...]
    xlh_ref[0] = xl[:, :H]
    xlh_ref[1] = xl[:, H:]
    rh_ref[0] = r[:, :H]
    rh_ref[1] = r[:, H:]


def _dense_kernel(x, W_lin, b_lin, root_emb, dinv_col):
    blk = 1000
    grid = N // blk
    return pl.pallas_call(
        _dense_body,
        grid=(grid,),
        in_specs=[
            pl.BlockSpec((blk, D), lambda j: (j, 0)),
            pl.BlockSpec((D, D), lambda j: (0, 0)),
            pl.BlockSpec((1, D), lambda j: (0, 0)),
            pl.BlockSpec((1, D), lambda j: (0, 0)),
            pl.BlockSpec((blk, 1), lambda j: (j, 0)),
        ],
        out_specs=[
            pl.BlockSpec((NC, blk, H), lambda j: (0, j, 0)),
            pl.BlockSpec((NC, blk, H), lambda j: (0, j, 0)),
        ],
        out_shape=[
            jax.ShapeDtypeStruct((NC, N, H), jnp.float32),
            jax.ShapeDtypeStruct((NC, N, H), jnp.float32),
        ],
    )(x, W_lin, b_lin.reshape(1, D), root_emb.reshape(1, D), dinv_col)


def _eh_body(ex_ref, w_ref, b_ref, ehh_ref):
    eh = jnp.dot(ex_ref[...], w_ref[...],
                 preferred_element_type=jnp.float32) + b_ref[...]
    ehh_ref[0] = eh[:, :H]
    ehh_ref[1] = eh[:, H:]


def _eh_kernel(ex_pad, W_edge_pad, b_edge):
    blk = 2000
    grid = E // blk
    return pl.pallas_call(
        _eh_body,
        grid=(grid,),
        in_specs=[
            pl.BlockSpec((blk, 8), lambda j: (j, 0)),
            pl.BlockSpec((8, D), lambda j: (0, 0)),
            pl.BlockSpec((1, D), lambda j: (0, 0)),
        ],
        out_specs=pl.BlockSpec((NC, blk, H), lambda j: (0, j, 0)),
        out_shape=jax.ShapeDtypeStruct((NC, E, H), jnp.float32),
    )(ex_pad, W_edge_pad, b_edge.reshape(1, D))


@jax.jit
def kernel(x, edge_index, ex, W_lin, b_lin, W_edge, b_edge, root_emb):
    src = edge_index[0]
    dst = edge_index[1]

    ex_pad = jnp.pad(ex, ((0, 0), (0, 1)))
    W_edge_pad = jnp.pad(W_edge, ((0, 1), (0, 0)))

    pdeg = _deg_kernel(src)
    norm, dinv = _norm_kernel(pdeg)
    dinv_col = dinv[:N].reshape(N, 1)
    xlh, rh = _dense_kernel(x, W_lin, b_lin, root_emb, dinv_col)
    ehh = _eh_kernel(ex_pad, W_edge_pad, b_edge)
    outh = _edge_kernel(xlh, ehh, rh, norm, src, dst)
    return jnp.concatenate([outh[0], outh[1]], axis=1)
